# Initial kernel scaffold; baseline (speedup 1.0000x reference)
import jax
import jax.numpy as jnp
from jax import lax
from jax.experimental import pallas as pl
from jax.experimental.pallas import tpu as pltpu
from jax.experimental.pallas import tpu_sc as plsc

N = 10000
NP = 10240
E = 320000
D = 128
DOUT = 200
NC = 2
NS = 16
NT = NC * NS
EPT = E // NT
CH = 80
NCHUNKS = EPT // CH
RPT = NP // NS
NB = 16
BR = NP // NB


def _sc_mesh():
    return plsc.VectorSubcoreMesh(core_axis_name="c", subcore_axis_name="s")


def _deg_body(row_hbm, col_hbm, w_hbm, degp_hbm,
              deg_sh, zero_v, si_v, di_v, w_v, wm_v):
    cid = lax.axis_index("c")
    sid = lax.axis_index("s")
    ebase = (cid * NS + sid) * EPT
    for i in range(RPT // 16):
        zero_v[pl.ds(i * 16, 16)] = jnp.zeros((16,), jnp.float32)
    rbase = pl.multiple_of(sid * RPT, 8)
    pltpu.sync_copy(zero_v, deg_sh.at[pl.ds(rbase, RPT)])
    plsc.subcore_barrier()

    def chunk(ci, carry):
        base = pl.multiple_of(ebase + ci * CH, 8)
        pltpu.sync_copy(row_hbm.at[pl.ds(base, CH)], si_v)
        pltpu.sync_copy(col_hbm.at[pl.ds(base, CH)], di_v)
        pltpu.sync_copy(w_hbm.at[pl.ds(base, CH)], w_v)
        for k in range(CH // 16):
            sl = pl.ds(k * 16, 16)
            s16 = si_v[sl]
            d16 = di_v[sl]
            w16 = w_v[sl]
            wm_v[sl] = jnp.where(s16 == d16, 0.0, w16)
        pltpu.sync_copy(wm_v, deg_sh.at[si_v], add=True)
        return carry

    lax.fori_loop(0, NCHUNKS, chunk, 0)
    plsc.subcore_barrier()
    pltpu.sync_copy(deg_sh.at[pl.ds(rbase, RPT)],
                    degp_hbm.at[cid, pl.ds(rbase, RPT)])


def _deg(row, col, w):
    return pl.kernel(
        _deg_body,
        out_type=jax.ShapeDtypeStruct((NC, NP), jnp.float32),
        mesh=_sc_mesh(),
        scratch_types=[
            pltpu.VMEM_SHARED((NP,), jnp.float32),
            pltpu.VMEM((RPT,), jnp.float32),
            pltpu.VMEM((CH,), jnp.int32),
            pltpu.VMEM((CH,), jnp.int32),
            pltpu.VMEM((CH,), jnp.float32),
            pltpu.VMEM((CH,), jnp.float32),
        ],
    )(row, col, w)


def _prop_body(row_hbm, col_hbm, w_hbm, dis_hbm, x_hbm, out_hbm,
               acc_sh, zrow_v, dis_v, si_v, di_v, w_v, nrm_v, rows_v, gsem):
    cid = lax.axis_index("c")
    sid = lax.axis_index("s")
    ebase = (cid * NS + sid) * EPT

    def zb(i, carry):
        for j in range(D // 16):
            zrow_v[i, pl.ds(j * 16, 16)] = jnp.zeros((16,), jnp.float32)
        return carry

    lax.fori_loop(0, 128, zb, 0)
    for b in range(RPT // 128):
        rb = pl.multiple_of(sid * RPT + b * 128, 8)
        pltpu.sync_copy(zrow_v, acc_sh.at[pl.ds(rb, 128)])
    pltpu.sync_copy(dis_hbm, dis_v)
    plsc.subcore_barrier()

    def chunk(ci, carry):
        base = pl.multiple_of(ebase + ci * CH, 8)
        pltpu.sync_copy(row_hbm.at[pl.ds(base, CH)], si_v)
        pltpu.sync_copy(col_hbm.at[pl.ds(base, CH)], di_v)
        pltpu.sync_copy(w_hbm.at[pl.ds(base, CH)], w_v)
        cp = pltpu.async_copy(x_hbm.at[si_v], rows_v, gsem)
        for k in range(CH // 16):
            sl = pl.ds(k * 16, 16)
            s16 = si_v[sl]
            d16 = di_v[sl]
            w16 = w_v[sl]
            a16 = plsc.load_gather(dis_v, [s16])
            b16 = plsc.load_gather(dis_v, [d16])
            wm = jnp.where(s16 == d16, 0.0, w16)
            nrm_v[sl] = -(a16 * wm * b16)
        cp.wait()

        def scale(e, c2):
            nv = nrm_v[e]
            for j in range(D // 16):
                sl = pl.ds(j * 16, 16)
                rows_v[e, sl] = rows_v[e, sl] * nv
            return c2

        lax.fori_loop(0, CH, scale, 0)
        pltpu.sync_copy(rows_v, acc_sh.at[di_v], add=True)
        return carry

    lax.fori_loop(0, NCHUNKS, chunk, 0)
    plsc.subcore_barrier()
    for b in range(RPT // 128):
        rb = pl.multiple_of(sid * RPT + b * 128, 8)
        pltpu.sync_copy(acc_sh.at[pl.ds(rb, 128)],
                        out_hbm.at[cid, pl.ds(rb, 128)])


def _prop(row, col, w, dis, xsrc):
    return pl.kernel(
        _prop_body,
        out_type=jax.ShapeDtypeStruct((NC, NP, D), jnp.float32),
        mesh=_sc_mesh(),
        scratch_types=[
            pltpu.VMEM_SHARED((NP, D), jnp.float32),
            pltpu.VMEM((128, D), jnp.float32),
            pltpu.VMEM((NP,), jnp.float32),
            pltpu.VMEM((CH,), jnp.int32),
            pltpu.VMEM((CH,), jnp.int32),
            pltpu.VMEM((CH,), jnp.float32),
            pltpu.VMEM((CH,), jnp.float32),
            pltpu.VMEM((CH, D), jnp.float32),
            pltpu.SemaphoreType.DMA,
        ],
    )(row, col, w, dis, xsrc)


def _prep_body(degp_ref, dis_ref):
    deg = degp_ref[0:1, :] + degp_ref[1:2, :]
    pos = deg > 0.0
    safe = jnp.where(pos, deg, 1.0)
    dis_ref[...] = jnp.where(pos, lax.rsqrt(safe), 0.0)


def _prep(degp):
    out = pl.pallas_call(
        _prep_body,
        out_shape=jax.ShapeDtypeStruct((1, NP), jnp.float32),
    )(degp)
    return out.reshape(NP)


def _ka_body(x_ref, p0_ref, p1_ref, w10_ref, b1_ref, w20_ref, w21_ref, b2_ref,
             tx1_ref, s1_ref, s2_ref):
    t1 = p0_ref[...] + p1_ref[...]
    tx1_ref[...] = t1
    xb = x_ref[...]
    s1_ref[...] = jnp.dot(xb, w10_ref[...],
                          preferred_element_type=jnp.float32) + b1_ref[...]
    s2_ref[...] = (jnp.dot(xb, w20_ref[...], preferred_element_type=jnp.float32)
                   + jnp.dot(t1, w21_ref[...], preferred_element_type=jnp.float32)
                   + b2_ref[...])


def _ka(xp, p0, p1, W1_0, b1, W2_0, W2_1, b2):
    row_spec = pl.BlockSpec((BR, D), lambda i: (i, 0))
    w_spec = pl.BlockSpec((D, DOUT), lambda i: (0, 0))
    b_spec = pl.BlockSpec((1, DOUT), lambda i: (0, 0))
    o_spec = pl.BlockSpec((BR, DOUT), lambda i: (i, 0))
    return pl.pallas_call(
        _ka_body,
        grid=(NB,),
        in_specs=[row_spec, row_spec, row_spec, w_spec, b_spec,
                  w_spec, w_spec, b_spec],
        out_specs=[row_spec, o_spec, o_spec],
        out_shape=[jax.ShapeDtypeStruct((NP, D), jnp.float32),
                   jax.ShapeDtypeStruct((NP, DOUT), jnp.float32),
                   jax.ShapeDtypeStruct((NP, DOUT), jnp.float32)],
    )(xp, p0, p1, W1_0, b1, W2_0, W2_1, b2)


def _kb_body(x_ref, t1_ref, u0_ref, u1_ref, w30_ref, w31_ref, w32_ref, b3_ref,
             s3_ref):
    xb = x_ref[...]
    t1 = t1_ref[...]
    tx2 = 2.0 * (u0_ref[...] + u1_ref[...]) - xb
    s3_ref[...] = (jnp.dot(xb, w30_ref[...], preferred_element_type=jnp.float32)
                   + jnp.dot(t1, w31_ref[...], preferred_element_type=jnp.float32)
                   + jnp.dot(tx2, w32_ref[...], preferred_element_type=jnp.float32)
                   + b3_ref[...])


def _kb(xp, tx1, u0, u1, W3_0, W3_1, W3_2, b3):
    row_spec = pl.BlockSpec((BR, D), lambda i: (i, 0))
    w_spec = pl.BlockSpec((D, DOUT), lambda i: (0, 0))
    b_spec = pl.BlockSpec((1, DOUT), lambda i: (0, 0))
    o_spec = pl.BlockSpec((BR, DOUT), lambda i: (i, 0))
    return pl.pallas_call(
        _kb_body,
        grid=(NB,),
        in_specs=[row_spec, row_spec, row_spec, row_spec,
                  w_spec, w_spec, w_spec, b_spec],
        out_specs=o_spec,
        out_shape=jax.ShapeDtypeStruct((NP, DOUT), jnp.float32),
    )(xp, tx1, u0, u1, W3_0, W3_1, W3_2, b3)


def kernel(x, edge_index, edge_weight, W1_0, b1, W2_0, W2_1, b2,
           W3_0, W3_1, W3_2, b3):
    row = edge_index[0]
    col = edge_index[1]
    w = edge_weight
    xp = jnp.zeros((NP, D), jnp.float32).at[:N, :].set(x)

    degp = _deg(row, col, w)
    dis = _prep(degp)
    p = _prop(row, col, w, dis, xp)
    tx1, s1, s2 = _ka(xp, p[0], p[1], W1_0, b1.reshape(1, DOUT),
                      W2_0, W2_1, b2.reshape(1, DOUT))
    u = _prop(row, col, w, dis, tx1)
    s3 = _kb(xp, tx1, u[0], u[1], W3_0, W3_1, W3_2, b3.reshape(1, DOUT))
    return s1[:N], s2[:N], s3[:N]


# speedup vs baseline: 6.3626x; 6.3626x over previous
import jax
import jax.numpy as jnp
from jax import lax
from jax.experimental import pallas as pl
from jax.experimental.pallas import tpu as pltpu
from jax.experimental.pallas import tpu_sc as plsc

N = 10000
NP = 10240
E = 320000
D = 128
DOUT = 200
NC = 2
NS = 16
NT = NC * NS
EPT = E // NT
CH = 80
NCHUNKS = EPT // CH
RPT = NP // NS
NB = 16
BR = NP // NB


def _sc_mesh():
    return plsc.VectorSubcoreMesh(core_axis_name="c", subcore_axis_name="s")


def _deg_body(row_hbm, col_hbm, w_hbm, degp_hbm,
              deg_sh, zero_v, si_v, di_v, w_v, wm_v):
    cid = lax.axis_index("c")
    sid = lax.axis_index("s")
    ebase = (cid * NS + sid) * EPT
    for i in range(RPT // 16):
        zero_v[pl.ds(i * 16, 16)] = jnp.zeros((16,), jnp.float32)
    rbase = pl.multiple_of(sid * RPT, 8)
    pltpu.sync_copy(zero_v, deg_sh.at[pl.ds(rbase, RPT)])
    plsc.subcore_barrier()

    def chunk(ci, carry):
        base = pl.multiple_of(ebase + ci * CH, 8)
        pltpu.sync_copy(row_hbm.at[pl.ds(base, CH)], si_v)
        pltpu.sync_copy(col_hbm.at[pl.ds(base, CH)], di_v)
        pltpu.sync_copy(w_hbm.at[pl.ds(base, CH)], w_v)
        for k in range(CH // 16):
            sl = pl.ds(k * 16, 16)
            s16 = si_v[sl]
            d16 = di_v[sl]
            w16 = w_v[sl]
            wm_v[sl] = jnp.where(s16 == d16, 0.0, w16)
        pltpu.sync_copy(wm_v, deg_sh.at[si_v], add=True)
        return carry

    lax.fori_loop(0, NCHUNKS, chunk, 0)
    plsc.subcore_barrier()
    pltpu.sync_copy(deg_sh.at[pl.ds(rbase, RPT)],
                    degp_hbm.at[cid, pl.ds(rbase, RPT)])


def _deg(row, col, w):
    return pl.kernel(
        _deg_body,
        out_type=jax.ShapeDtypeStruct((NC, NP), jnp.float32),
        mesh=_sc_mesh(),
        scratch_types=[
            pltpu.VMEM_SHARED((NP,), jnp.float32),
            pltpu.VMEM((RPT,), jnp.float32),
            pltpu.VMEM((CH,), jnp.int32),
            pltpu.VMEM((CH,), jnp.int32),
            pltpu.VMEM((CH,), jnp.float32),
            pltpu.VMEM((CH,), jnp.float32),
        ],
    )(row, col, w)


def _prop_body(row_hbm, col_hbm, w_hbm, dis_hbm, x_hbm, out_hbm,
               acc_sh, zrow_v, dis_v, si_v, di_v, w_v, rows_v, gsem):
    cid = lax.axis_index("c")
    sid = lax.axis_index("s")
    ebase = (cid * NS + sid) * EPT

    def zb(i, carry):
        for j in range(D // 16):
            zrow_v[i, pl.ds(j * 16, 16)] = jnp.zeros((16,), jnp.float32)
        return carry

    lax.fori_loop(0, 128, zb, 0)
    for b in range(RPT // 128):
        rb = pl.multiple_of(sid * RPT + b * 128, 8)
        pltpu.sync_copy(zrow_v, acc_sh.at[pl.ds(rb, 128)])
    pltpu.sync_copy(dis_hbm, dis_v)
    plsc.subcore_barrier()

    def chunk(ci, carry):
        base = pl.multiple_of(ebase + ci * CH, 8)
        pltpu.sync_copy(row_hbm.at[pl.ds(base, CH)], si_v)
        pltpu.sync_copy(col_hbm.at[pl.ds(base, CH)], di_v)
        pltpu.sync_copy(w_hbm.at[pl.ds(base, CH)], w_v)
        cp = pltpu.async_copy(x_hbm.at[si_v], rows_v, gsem)
        nrm16 = []
        for k in range(CH // 16):
            sl = pl.ds(k * 16, 16)
            s16 = si_v[sl]
            d16 = di_v[sl]
            w16 = w_v[sl]
            a16 = plsc.load_gather(dis_v, [s16])
            b16 = plsc.load_gather(dis_v, [d16])
            wm = jnp.where(s16 == d16, 0.0, w16)
            nrm16.append(-(a16 * wm * b16))
        cp.wait()
        for k in range(CH // 16):
            n16 = nrm16[k]
            for l in range(16):
                e = k * 16 + l
                nv = n16[l]
                for j in range(D // 16):
                    sl = pl.ds(j * 16, 16)
                    rows_v[e, sl] = rows_v[e, sl] * nv
        pltpu.sync_copy(rows_v, acc_sh.at[di_v], add=True)
        return carry

    lax.fori_loop(0, NCHUNKS, chunk, 0)
    plsc.subcore_barrier()
    for b in range(RPT // 128):
        rb = pl.multiple_of(sid * RPT + b * 128, 8)
        pltpu.sync_copy(acc_sh.at[pl.ds(rb, 128)],
                        out_hbm.at[cid, pl.ds(rb, 128)])


def _prop(row, col, w, dis, xsrc):
    return pl.kernel(
        _prop_body,
        out_type=jax.ShapeDtypeStruct((NC, NP, D), jnp.float32),
        mesh=_sc_mesh(),
        scratch_types=[
            pltpu.VMEM_SHARED((NP, D), jnp.float32),
            pltpu.VMEM((128, D), jnp.float32),
            pltpu.VMEM((NP,), jnp.float32),
            pltpu.VMEM((CH,), jnp.int32),
            pltpu.VMEM((CH,), jnp.int32),
            pltpu.VMEM((CH,), jnp.float32),
            pltpu.VMEM((CH, D), jnp.float32),
            pltpu.SemaphoreType.DMA,
        ],
        compiler_params=pltpu.CompilerParams(needs_layout_passes=False),
    )(row, col, w, dis, xsrc)


def _prep_body(degp_ref, dis_ref):
    deg = degp_ref[0:1, :] + degp_ref[1:2, :]
    pos = deg > 0.0
    safe = jnp.where(pos, deg, 1.0)
    dis_ref[...] = jnp.where(pos, lax.rsqrt(safe), 0.0)


def _prep(degp):
    out = pl.pallas_call(
        _prep_body,
        out_shape=jax.ShapeDtypeStruct((1, NP), jnp.float32),
    )(degp)
    return out.reshape(NP)


def _ka_body(x_ref, p0_ref, p1_ref, w10_ref, b1_ref, w20_ref, w21_ref, b2_ref,
             tx1_ref, s1_ref, s2_ref):
    t1 = p0_ref[...] + p1_ref[...]
    tx1_ref[...] = t1
    xb = x_ref[...]
    s1_ref[...] = jnp.dot(xb, w10_ref[...],
                          preferred_element_type=jnp.float32) + b1_ref[...]
    s2_ref[...] = (jnp.dot(xb, w20_ref[...], preferred_element_type=jnp.float32)
                   + jnp.dot(t1, w21_ref[...], preferred_element_type=jnp.float32)
                   + b2_ref[...])


def _ka(xp, p0, p1, W1_0, b1, W2_0, W2_1, b2):
    row_spec = pl.BlockSpec((BR, D), lambda i: (i, 0))
    w_spec = pl.BlockSpec((D, DOUT), lambda i: (0, 0))
    b_spec = pl.BlockSpec((1, DOUT), lambda i: (0, 0))
    o_spec = pl.BlockSpec((BR, DOUT), lambda i: (i, 0))
    return pl.pallas_call(
        _ka_body,
        grid=(NB,),
        in_specs=[row_spec, row_spec, row_spec, w_spec, b_spec,
                  w_spec, w_spec, b_spec],
        out_specs=[row_spec, o_spec, o_spec],
        out_shape=[jax.ShapeDtypeStruct((NP, D), jnp.float32),
                   jax.ShapeDtypeStruct((NP, DOUT), jnp.float32),
                   jax.ShapeDtypeStruct((NP, DOUT), jnp.float32)],
    )(xp, p0, p1, W1_0, b1, W2_0, W2_1, b2)


def _kb_body(x_ref, t1_ref, u0_ref, u1_ref, w30_ref, w31_ref, w32_ref, b3_ref,
             s3_ref):
    xb = x_ref[...]
    t1 = t1_ref[...]
    tx2 = 2.0 * (u0_ref[...] + u1_ref[...]) - xb
    s3_ref[...] = (jnp.dot(xb, w30_ref[...], preferred_element_type=jnp.float32)
                   + jnp.dot(t1, w31_ref[...], preferred_element_type=jnp.float32)
                   + jnp.dot(tx2, w32_ref[...], preferred_element_type=jnp.float32)
                   + b3_ref[...])


def _kb(xp, tx1, u0, u1, W3_0, W3_1, W3_2, b3):
    row_spec = pl.BlockSpec((BR, D), lambda i: (i, 0))
    w_spec = pl.BlockSpec((D, DOUT), lambda i: (0, 0))
    b_spec = pl.BlockSpec((1, DOUT), lambda i: (0, 0))
    o_spec = pl.BlockSpec((BR, DOUT), lambda i: (i, 0))
    return pl.pallas_call(
        _kb_body,
        grid=(NB,),
        in_specs=[row_spec, row_spec, row_spec, row_spec,
                  w_spec, w_spec, w_spec, b_spec],
        out_specs=o_spec,
        out_shape=jax.ShapeDtypeStruct((NP, DOUT), jnp.float32),
    )(xp, tx1, u0, u1, W3_0, W3_1, W3_2, b3)


def kernel(x, edge_index, edge_weight, W1_0, b1, W2_0, W2_1, b2,
           W3_0, W3_1, W3_2, b3):
    row = edge_index[0]
    col = edge_index[1]
    w = edge_weight
    xp = jnp.zeros((NP, D), jnp.float32).at[:N, :].set(x)

    degp = _deg(row, col, w)
    dis = _prep(degp)
    p = _prop(row, col, w, dis, xp)
    tx1, s1, s2 = _ka(xp, p[0], p[1], W1_0, b1.reshape(1, DOUT),
                      W2_0, W2_1, b2.reshape(1, DOUT))
    u = _prop(row, col, w, dis, tx1)
    s3 = _kb(xp, tx1, u[0], u[1], W3_0, W3_1, W3_2, b3.reshape(1, DOUT))
    return s1[:N], s2[:N], s3[:N]


# speedup vs baseline: 15.0051x; 2.3583x over previous
import jax
import jax.numpy as jnp
from jax import lax
from jax.experimental import pallas as pl
from jax.experimental.pallas import tpu as pltpu
from jax.experimental.pallas import tpu_sc as plsc

N = 10000
NP = 10240
E = 320000
D = 128
DOUT = 200
NC = 2
NS = 16
NT = NC * NS
EPT = E // NT
CH = 80
NCHUNKS = EPT // CH
RPT = NP // NS
NB = 16
BR = NP // NB


def _sc_mesh():
    return plsc.VectorSubcoreMesh(core_axis_name="c", subcore_axis_name="s")


def _deg_body(edata_hbm, degp_hbm,
              deg_sh, zero_v, blk0, blk1, blk2, blk3, wm0, wm1, wm2, wm3,
              is0, is1, is2, is3, ss0, ss1, ss2, ss3):
    cid = lax.axis_index("c")
    sid = lax.axis_index("s")
    tid = cid * NS + sid
    blks = (blk0, blk1, blk2, blk3)
    wms = (wm0, wm1, wm2, wm3)
    isems = (is0, is1, is2, is3)
    ssems = (ss0, ss1, ss2, ss3)

    for i in range(RPT // 16):
        zero_v[pl.ds(i * 16, 16)] = jnp.zeros((16,), jnp.float32)
    rbase = pl.multiple_of(sid * RPT, 8)
    pltpu.sync_copy(zero_v, deg_sh.at[pl.ds(rbase, RPT)])
    plsc.subcore_barrier()

    def i_start(c, r4):
        pltpu.async_copy(edata_hbm.at[tid, c], blks[r4], isems[r4])

    def i_wait(r4):
        pltpu.make_async_copy(edata_hbm.at[tid, 0], blks[r4], isems[r4]).wait()

    def s_start(r4):
        pltpu.async_copy(wms[r4], deg_sh.at[blks[r4].at[0]], ssems[r4],
                         add=True)

    def s_wait(r4):
        pltpu.make_async_copy(wms[r4], deg_sh.at[blks[r4].at[0]],
                              ssems[r4]).wait()

    def compute(r4):
        blk = blks[r4]
        wm = wms[r4]

        def kbody(k, carry):
            sl = pl.ds(k * 16, 16)
            s16 = blk[0, sl]
            d16 = blk[1, sl]
            w16 = plsc.bitcast(blk[2, sl], jnp.float32)
            wm[pl.ds(k * 16, 16)] = jnp.where(s16 == d16, 0.0, w16)
            return carry

        lax.fori_loop(0, CH // 16, kbody, 0)

    def step(c, r4, do_swait, do_istart):
        i_wait(r4)
        compute(r4)
        s_start(r4)
        if do_swait:
            s_wait((r4 + 2) % 4)
        if do_istart:
            i_start(c + 2, (r4 + 2) % 4)

    i_start(0, 0)
    i_start(1, 1)
    step(0, 0, False, True)
    step(1, 1, False, True)

    def quad(pp, carry):
        c = 2 + 4 * pp
        for q in range(4):
            step(c + q, (2 + q) % 4, True, True)
        return carry

    lax.fori_loop(0, (NCHUNKS - 5) // 4, quad, 0)
    for c in range(NCHUNKS - 3, NCHUNKS):
        step(c, c % 4, True, c + 2 < NCHUNKS)
    s_wait((NCHUNKS - 2) % 4)
    s_wait((NCHUNKS - 1) % 4)
    plsc.subcore_barrier()
    pltpu.sync_copy(deg_sh.at[pl.ds(rbase, RPT)],
                    degp_hbm.at[cid, pl.ds(rbase, RPT)])


def _deg(edata):
    return pl.kernel(
        _deg_body,
        out_type=jax.ShapeDtypeStruct((NC, NP), jnp.float32),
        mesh=_sc_mesh(),
        scratch_types=(
            [pltpu.VMEM_SHARED((NP,), jnp.float32),
             pltpu.VMEM((RPT,), jnp.float32)]
            + [pltpu.VMEM((3, CH), jnp.int32) for _ in range(4)]
            + [pltpu.VMEM((CH,), jnp.float32) for _ in range(4)]
            + [pltpu.SemaphoreType.DMA for _ in range(8)]
        ),
        compiler_params=pltpu.CompilerParams(needs_layout_passes=False),
    )(edata)


def _prop_body(edata_hbm, dis_hbm, x_hbm, out_hbm,
               acc_sh, dis_v, blk0, blk1, blk2, blk3, rows0, rows1, rows2,
               is0, is1, is2, is3, gs0, gs1, gs2, ss0, ss1, ss2, stsem):
    cid = lax.axis_index("c")
    sid = lax.axis_index("s")
    tid = cid * NS + sid
    blks = (blk0, blk1, blk2, blk3)
    rows = (rows0, rows1, rows2)
    isems = (is0, is1, is2, is3)
    gsems = (gs0, gs1, gs2)
    ssems = (ss0, ss1, ss2)

    pltpu.async_copy(dis_hbm, dis_v, stsem)

    def zb(i, carry):
        for j in range(D // 16):
            rows0[i, pl.ds(j * 16, 16)] = jnp.zeros((16,), jnp.float32)
        return carry

    lax.fori_loop(0, CH, zb, 0)
    for b in range(RPT // CH):
        rb = pl.multiple_of(sid * RPT + b * CH, 8)
        pltpu.sync_copy(rows0, acc_sh.at[pl.ds(rb, CH)])
    pltpu.make_async_copy(dis_hbm, dis_v, stsem).wait()
    plsc.subcore_barrier()

    def i_start(c, r4):
        pltpu.async_copy(edata_hbm.at[tid, c], blks[r4], isems[r4])

    def i_wait(r4):
        pltpu.make_async_copy(edata_hbm.at[tid, 0], blks[r4], isems[r4]).wait()

    def g_start(r4, r3):
        pltpu.async_copy(x_hbm.at[blks[r4].at[0]], rows[r3], gsems[r3])

    def g_wait(r3):
        pltpu.make_async_copy(x_hbm.at[blks[0].at[0]], rows[r3],
                              gsems[r3]).wait()

    def s_start(r4, r3):
        pltpu.async_copy(rows[r3], acc_sh.at[blks[r4].at[1]], ssems[r3],
                         add=True)

    def s_wait(r3):
        pltpu.make_async_copy(rows[r3], acc_sh.at[blks[0].at[1]],
                              ssems[r3]).wait()

    def scale(r4, r3):
        blk = blks[r4]
        buf = rows[r3]

        def kbody(k, carry):
            sl = pl.ds(k * 16, 16)
            s16 = blk[0, sl]
            d16 = blk[1, sl]
            w16 = plsc.bitcast(blk[2, sl], jnp.float32)
            a16 = plsc.load_gather(dis_v, [s16])
            b16 = plsc.load_gather(dis_v, [d16])
            wm = jnp.where(s16 == d16, 0.0, w16)
            n16 = -(a16 * wm * b16)
            e0 = k * 16
            for l in range(16):
                nv = n16[l]
                for j in range(D // 16):
                    fs = pl.ds(j * 16, 16)
                    buf[e0 + l, fs] = buf[e0 + l, fs] * nv
            return carry

        lax.fori_loop(0, CH // 16, kbody, 0)

    def step(c, r3, r4, do_swait, do_istart, do_gnext):
        g_wait(r3)
        scale(r4, r3)
        s_start(r4, r3)
        if do_swait:
            s_wait((r3 + 2) % 3)
        if do_istart:
            i_start(c + 3, (r4 + 3) % 4)
        if do_gnext:
            i_wait((r4 + 2) % 4)
            g_start((r4 + 2) % 4, (r3 + 2) % 3)

    i_start(0, 0)
    i_start(1, 1)
    i_start(2, 2)
    i_wait(0)
    g_start(0, 0)
    i_wait(1)
    g_start(1, 1)
    step(0, 0, 0, False, True, True)
    step(1, 1, 1, True, True, True)

    def twelve(pp, carry):
        c = 2 + 12 * pp
        for q in range(12):
            step(c + q, (2 + q) % 3, (2 + q) % 4, True, True, True)
        return carry

    lax.fori_loop(0, (NCHUNKS - 5) // 12, twelve, 0)
    for c in range(NCHUNKS - 3, NCHUNKS):
        step(c, c % 3, c % 4, True, c + 3 < NCHUNKS, c + 2 < NCHUNKS)
    s_wait((NCHUNKS - 1) % 3)
    plsc.subcore_barrier()
    for b in range(RPT // 128):
        rb = pl.multiple_of(sid * RPT + b * 128, 8)
        pltpu.async_copy(acc_sh.at[pl.ds(rb, 128)],
                         out_hbm.at[cid, pl.ds(rb, 128)], stsem)
    for b in range(RPT // 128):
        rb = pl.multiple_of(sid * RPT + b * 128, 8)
        pltpu.make_async_copy(acc_sh.at[pl.ds(rb, 128)],
                              out_hbm.at[cid, pl.ds(rb, 128)], stsem).wait()


def _prop(edata, dis, xsrc):
    return pl.kernel(
        _prop_body,
        out_type=jax.ShapeDtypeStruct((NC, NP, D), jnp.float32),
        mesh=_sc_mesh(),
        scratch_types=(
            [pltpu.VMEM_SHARED((NP, D), jnp.float32),
             pltpu.VMEM((NP,), jnp.float32)]
            + [pltpu.VMEM((3, CH), jnp.int32) for _ in range(4)]
            + [pltpu.VMEM((CH, D), jnp.float32) for _ in range(3)]
            + [pltpu.SemaphoreType.DMA for _ in range(11)]
        ),
        compiler_params=pltpu.CompilerParams(needs_layout_passes=False),
    )(edata, dis, xsrc)


def _prep_body(degp_ref, dis_ref):
    deg = degp_ref[0:1, :] + degp_ref[1:2, :]
    pos = deg > 0.0
    safe = jnp.where(pos, deg, 1.0)
    dis_ref[...] = jnp.where(pos, lax.rsqrt(safe), 0.0)


def _prep(degp):
    out = pl.pallas_call(
        _prep_body,
        out_shape=jax.ShapeDtypeStruct((1, NP), jnp.float32),
    )(degp)
    return out.reshape(NP)


def _ka_body(x_ref, p0_ref, p1_ref, w10_ref, b1_ref, w20_ref, w21_ref, b2_ref,
             tx1_ref, s1_ref, s2_ref):
    t1 = p0_ref[...] + p1_ref[...]
    tx1_ref[...] = t1
    xb = x_ref[...]
    s1_ref[...] = jnp.dot(xb, w10_ref[...],
                          preferred_element_type=jnp.float32) + b1_ref[...]
    s2_ref[...] = (jnp.dot(xb, w20_ref[...], preferred_element_type=jnp.float32)
                   + jnp.dot(t1, w21_ref[...], preferred_element_type=jnp.float32)
                   + b2_ref[...])


def _ka(xp, p0, p1, W1_0, b1, W2_0, W2_1, b2):
    row_spec = pl.BlockSpec((BR, D), lambda i: (i, 0))
    w_spec = pl.BlockSpec((D, DOUT), lambda i: (0, 0))
    b_spec = pl.BlockSpec((1, DOUT), lambda i: (0, 0))
    o_spec = pl.BlockSpec((BR, DOUT), lambda i: (i, 0))
    return pl.pallas_call(
        _ka_body,
        grid=(NB,),
        in_specs=[row_spec, row_spec, row_spec, w_spec, b_spec,
                  w_spec, w_spec, b_spec],
        out_specs=[row_spec, o_spec, o_spec],
        out_shape=[jax.ShapeDtypeStruct((NP, D), jnp.float32),
                   jax.ShapeDtypeStruct((NP, DOUT), jnp.float32),
                   jax.ShapeDtypeStruct((NP, DOUT), jnp.float32)],
    )(xp, p0, p1, W1_0, b1, W2_0, W2_1, b2)


def _kb_body(x_ref, t1_ref, u0_ref, u1_ref, w30_ref, w31_ref, w32_ref, b3_ref,
             s3_ref):
    xb = x_ref[...]
    t1 = t1_ref[...]
    tx2 = 2.0 * (u0_ref[...] + u1_ref[...]) - xb
    s3_ref[...] = (jnp.dot(xb, w30_ref[...], preferred_element_type=jnp.float32)
                   + jnp.dot(t1, w31_ref[...], preferred_element_type=jnp.float32)
                   + jnp.dot(tx2, w32_ref[...], preferred_element_type=jnp.float32)
                   + b3_ref[...])


def _kb(xp, tx1, u0, u1, W3_0, W3_1, W3_2, b3):
    row_spec = pl.BlockSpec((BR, D), lambda i: (i, 0))
    w_spec = pl.BlockSpec((D, DOUT), lambda i: (0, 0))
    b_spec = pl.BlockSpec((1, DOUT), lambda i: (0, 0))
    o_spec = pl.BlockSpec((BR, DOUT), lambda i: (i, 0))
    return pl.pallas_call(
        _kb_body,
        grid=(NB,),
        in_specs=[row_spec, row_spec, row_spec, row_spec,
                  w_spec, w_spec, w_spec, b_spec],
        out_specs=o_spec,
        out_shape=jax.ShapeDtypeStruct((NP, DOUT), jnp.float32),
    )(xp, tx1, u0, u1, W3_0, W3_1, W3_2, b3)


def kernel(x, edge_index, edge_weight, W1_0, b1, W2_0, W2_1, b2,
           W3_0, W3_1, W3_2, b3):
    row = edge_index[0]
    col = edge_index[1]
    w = edge_weight
    xp = jnp.zeros((NP, D), jnp.float32).at[:N, :].set(x)

    w_bits = lax.bitcast_convert_type(w, jnp.int32)
    edata = jnp.stack([row, col, w_bits], axis=0)
    edata = edata.reshape(3, NT, NCHUNKS, CH).transpose(1, 2, 0, 3)

    degp = _deg(edata)
    dis = _prep(degp)
    p = _prop(edata, dis, xp)
    tx1, s1, s2 = _ka(xp, p[0], p[1], W1_0, b1.reshape(1, DOUT),
                      W2_0, W2_1, b2.reshape(1, DOUT))
    u = _prop(edata, dis, tx1)
    s3 = _kb(xp, tx1, u[0], u[1], W3_0, W3_1, W3_2, b3.reshape(1, DOUT))
    return s1[:N], s2[:N], s3[:N]


# speedup vs baseline: 15.6510x; 1.0430x over previous
import jax
import jax.numpy as jnp
from jax import lax
from jax.experimental import pallas as pl
from jax.experimental.pallas import tpu as pltpu
from jax.experimental.pallas import tpu_sc as plsc

N = 10000
NP = 10240
E = 320000
D = 128
DOUT = 200
NC = 2
NS = 16
NT = NC * NS
EPT = E // NT
CH = 80
NCHUNKS = EPT // CH
TOTCH = E // CH
DCH = TOTCH // NS
RPT = NP // NS
NB = 25
BR = N // NB


def _sc_mesh():
    return plsc.VectorSubcoreMesh(core_axis_name="c", subcore_axis_name="s")


def _degnorm_body(rowc_hbm, colc_hbm, wc_hbm, norm_hbm,
                  deg_sh, dis_sh, degv, nrm_st, t640, a640, d640,
                  si0, si1, si2, si3, co0, co1, co2, co3, w0, w1, w2, w3,
                  is0, is1, is2, is3):
    cid = lax.axis_index("c")
    sid = lax.axis_index("s")
    tid = cid * NS + sid
    sis = (si0, si1, si2, si3)
    cos = (co0, co1, co2, co3)
    ws = (w0, w1, w2, w3)
    isems = (is0, is1, is2, is3)

    def i_start(fc, r4):
        pltpu.async_copy(rowc_hbm.at[fc], sis[r4], isems[r4])
        pltpu.async_copy(colc_hbm.at[fc], cos[r4], isems[r4])
        pltpu.async_copy(wc_hbm.at[fc], ws[r4], isems[r4])

    def i_wait(r4):
        for _ in range(3):
            pltpu.make_async_copy(rowc_hbm.at[0], sis[r4], isems[r4]).wait()

    def zv(i, carry):
        degv[pl.ds(i * 16, 16)] = jnp.zeros((16,), jnp.float32)
        return carry

    lax.fori_loop(0, NP // 16, zv, 0)

    dbase = sid * DCH

    def acc_chunk(r4):
        i_wait(r4)
        blk_s = sis[r4]
        blk_d = cos[r4]
        blk_w = ws[r4]

        def kbody(k, carry):
            sl = pl.ds(k * 16, 16)
            s16 = blk_s[sl]
            d16 = blk_d[sl]
            w16 = blk_w[sl]
            wm = jnp.where(s16 == d16, 0.0, w16)
            plsc.addupdate_scatter(degv, [s16], wm)
            return carry

        lax.fori_loop(0, CH // 16, kbody, 0)

    i_start(dbase + 0, 0)
    i_start(dbase + 1, 1)

    def aquad(pp, carry):
        ch = 4 * pp
        for q in range(4):
            acc_chunk(q)
            i_start(dbase + ch + q + 2, (q + 2) % 4)
        return carry

    lax.fori_loop(0, (DCH - 2) // 4, aquad, 0)
    acc_chunk((DCH - 2) % 4)
    acc_chunk((DCH - 1) % 4)
    pltpu.sync_copy(degv, deg_sh.at[sid])
    plsc.subcore_barrier()

    rbase = pl.multiple_of(sid * RPT, 8)
    pltpu.sync_copy(deg_sh.at[0, pl.ds(rbase, RPT)], a640)
    for t in range(1, NS):
        pltpu.sync_copy(deg_sh.at[t, pl.ds(rbase, RPT)], t640)

        def addb(g, carry):
            sl = pl.ds(g * 16, 16)
            a640[sl] = a640[sl] + t640[sl]
            return carry

        lax.fori_loop(0, RPT // 16, addb, 0)

    def disb(g, carry):
        sl = pl.ds(g * 16, 16)
        x16 = a640[sl]
        bits = plsc.bitcast(x16, jnp.int32)
        y = plsc.bitcast(jnp.int32(0x5F3759DF) - (bits >> 1), jnp.float32)
        for _ in range(3):
            y = y * (1.5 - 0.5 * x16 * y * y)
        d640[sl] = jnp.where(x16 > 0.0, y, 0.0)
        return carry

    lax.fori_loop(0, RPT // 16, disb, 0)
    pltpu.sync_copy(d640, dis_sh.at[pl.ds(rbase, RPT)])
    plsc.subcore_barrier()
    pltpu.sync_copy(dis_sh, degv)

    nbase = tid * NCHUNKS

    def nrm_chunk(ch, r4):
        i_wait(r4)
        blk_s = sis[r4]
        blk_d = cos[r4]
        blk_w = ws[r4]

        def kbody(k, carry):
            sl = pl.ds(k * 16, 16)
            s16 = blk_s[sl]
            d16 = blk_d[sl]
            w16 = blk_w[sl]
            a16 = plsc.load_gather(degv, [s16])
            b16 = plsc.load_gather(degv, [d16])
            wm = jnp.where(s16 == d16, 0.0, w16)
            nrm_st[pl.ds(ch * CH + k * 16, 16)] = -(a16 * wm * b16)
            return carry

        lax.fori_loop(0, CH // 16, kbody, 0)

    i_start(nbase + 0, 0)
    i_start(nbase + 1, 1)

    def cquad(pp, carry):
        ch = 4 * pp
        for q in range(4):
            nrm_chunk(ch + q, q)
            i_start(nbase + ch + q + 2, (q + 2) % 4)
        return carry

    lax.fori_loop(0, (NCHUNKS - 5) // 4, cquad, 0)
    for ch in range(NCHUNKS - 5, NCHUNKS):
        nrm_chunk(ch, ch % 4)
        if ch + 2 < NCHUNKS:
            i_start(nbase + ch + 2, (ch + 2) % 4)
    pltpu.sync_copy(nrm_st, norm_hbm.at[tid])


def _degnorm(rowc, colc, wc):
    return pl.kernel(
        _degnorm_body,
        out_type=jax.ShapeDtypeStruct((NT, EPT), jnp.float32),
        mesh=_sc_mesh(),
        scratch_types=(
            [pltpu.VMEM_SHARED((NS, NP), jnp.float32),
             pltpu.VMEM_SHARED((NP,), jnp.float32),
             pltpu.VMEM((NP,), jnp.float32),
             pltpu.VMEM((EPT,), jnp.float32),
             pltpu.VMEM((RPT,), jnp.float32),
             pltpu.VMEM((RPT,), jnp.float32),
             pltpu.VMEM((RPT,), jnp.float32)]
            + [pltpu.VMEM((CH,), jnp.int32) for _ in range(8)]
            + [pltpu.VMEM((CH,), jnp.float32) for _ in range(4)]
            + [pltpu.SemaphoreType.DMA for _ in range(4)]
        ),
        compiler_params=pltpu.CompilerParams(needs_layout_passes=False),
    )(rowc, colc, wc)


def _prop_body(rowc_hbm, colc_hbm, nrmc_hbm, x_hbm, out_hbm,
               acc_sh, rows0, rows1, rows2, rows3,
               si0, si1, si2, si3, di0, di1, di2, di3, nr0, nr1, nr2, nr3,
               is0, is1, is2, is3, ds0, ds1, ds2, ds3,
               gs0, gs1, gs2, gs3, ss0, ss1, ss2, ss3, stsem):
    cid = lax.axis_index("c")
    sid = lax.axis_index("s")
    tid = cid * NS + sid
    rows = (rows0, rows1, rows2, rows3)
    sis = (si0, si1, si2, si3)
    dis_ = (di0, di1, di2, di3)
    nrs = (nr0, nr1, nr2, nr3)
    isems = (is0, is1, is2, is3)
    dsems = (ds0, ds1, ds2, ds3)
    gsems = (gs0, gs1, gs2, gs3)
    ssems = (ss0, ss1, ss2, ss3)
    cbase = tid * NCHUNKS

    def zb(i, carry):
        for j in range(D // 16):
            rows0[i, pl.ds(j * 16, 16)] = jnp.zeros((16,), jnp.float32)
        return carry

    lax.fori_loop(0, CH, zb, 0)
    for b in range(RPT // CH):
        rb = pl.multiple_of(sid * RPT + b * CH, 8)
        pltpu.sync_copy(rows0, acc_sh.at[pl.ds(rb, CH)])
    plsc.subcore_barrier()

    def si_start(c, r4):
        pltpu.async_copy(rowc_hbm.at[cbase + c], sis[r4], isems[r4])
        pltpu.async_copy(nrmc_hbm.at[cbase + c], nrs[r4], isems[r4])

    def si_wait(r4):
        for _ in range(2):
            pltpu.make_async_copy(rowc_hbm.at[0], sis[r4], isems[r4]).wait()

    def di_start(c, r4):
        pltpu.async_copy(colc_hbm.at[cbase + c], dis_[r4], dsems[r4])

    def di_wait(r4):
        pltpu.make_async_copy(colc_hbm.at[0], dis_[r4], dsems[r4]).wait()

    def g_start(r4):
        pltpu.async_copy(x_hbm.at[sis[r4]], rows[r4], gsems[r4])

    def g_wait(r4):
        pltpu.make_async_copy(x_hbm.at[sis[0]], rows[r4], gsems[r4]).wait()

    def s_start(r4):
        pltpu.async_copy(rows[r4], acc_sh.at[dis_[r4]], ssems[r4], add=True)

    def s_wait(r4):
        pltpu.make_async_copy(rows[0], acc_sh.at[dis_[0]], ssems[r4]).wait()

    def scale(r4):
        buf = rows[r4]
        nrm = nrs[r4]

        def kbody(k, carry):
            n16 = nrm[pl.ds(k * 16, 16)]
            e0 = k * 16
            for l in range(16):
                nv = n16[l]
                for j in range(D // 16):
                    fs = pl.ds(j * 16, 16)
                    buf[e0 + l, fs] = buf[e0 + l, fs] * nv
            return carry

        lax.fori_loop(0, CH // 16, kbody, 0)

    def step(c, r4, do_swait, do_si, do_di, do_g):
        g_wait(r4)
        scale(r4)
        di_wait(r4)
        s_start(r4)
        if do_swait:
            s_wait((r4 + 2) % 4)
        if do_si:
            si_start(c + 3, (r4 + 3) % 4)
        if do_di:
            di_start(c + 2, (r4 + 2) % 4)
        if do_g:
            si_wait((r4 + 2) % 4)
            g_start((r4 + 2) % 4)

    si_start(0, 0)
    si_start(1, 1)
    si_start(2, 2)
    di_start(0, 0)
    di_start(1, 1)
    si_wait(0)
    g_start(0)
    si_wait(1)
    g_start(1)
    step(0, 0, False, True, True, True)
    step(1, 1, False, True, True, True)

    def quad(pp, carry):
        c = 2 + 4 * pp
        for q in range(4):
            step(c + q, (2 + q) % 4, True, True, True, True)
        return carry

    lax.fori_loop(0, (NCHUNKS - 5) // 4, quad, 0)
    for c in range(NCHUNKS - 3, NCHUNKS):
        step(c, c % 4, True, c + 3 < NCHUNKS, c + 2 < NCHUNKS,
             c + 2 < NCHUNKS)
    s_wait((NCHUNKS - 2) % 4)
    s_wait((NCHUNKS - 1) % 4)
    plsc.subcore_barrier()
    for b in range(RPT // 128):
        rb = pl.multiple_of(sid * RPT + b * 128, 8)
        pltpu.async_copy(acc_sh.at[pl.ds(rb, 128)],
                         out_hbm.at[cid, pl.ds(rb, 128)], stsem)
    for b in range(RPT // 128):
        rb = pl.multiple_of(sid * RPT + b * 128, 8)
        pltpu.make_async_copy(acc_sh.at[pl.ds(rb, 128)],
                              out_hbm.at[cid, pl.ds(rb, 128)], stsem).wait()


def _prop(rowc, colc, nrmc, xsrc):
    return pl.kernel(
        _prop_body,
        out_type=jax.ShapeDtypeStruct((NC, NP, D), jnp.float32),
        mesh=_sc_mesh(),
        scratch_types=(
            [pltpu.VMEM_SHARED((NP, D), jnp.float32)]
            + [pltpu.VMEM((CH, D), jnp.float32) for _ in range(4)]
            + [pltpu.VMEM((CH,), jnp.int32) for _ in range(8)]
            + [pltpu.VMEM((CH,), jnp.float32) for _ in range(4)]
            + [pltpu.SemaphoreType.DMA for _ in range(17)]
        ),
        compiler_params=pltpu.CompilerParams(needs_layout_passes=False),
    )(rowc, colc, nrmc, xsrc)


def _ka_body(x_ref, p0_ref, p1_ref, w10_ref, b1_ref, w20_ref, w21_ref, b2_ref,
             tx1_ref, s1_ref, s2_ref):
    t1 = p0_ref[...] + p1_ref[...]
    tx1_ref[...] = t1
    xb = x_ref[...]
    s1_ref[...] = jnp.dot(xb, w10_ref[...],
                          preferred_element_type=jnp.float32) + b1_ref[...]
    s2_ref[...] = (jnp.dot(xb, w20_ref[...], preferred_element_type=jnp.float32)
                   + jnp.dot(t1, w21_ref[...], preferred_element_type=jnp.float32)
                   + b2_ref[...])


def _ka(x, p0, p1, W1_0, b1, W2_0, W2_1, b2):
    row_spec = pl.BlockSpec((BR, D), lambda i: (i, 0))
    w_spec = pl.BlockSpec((D, DOUT), lambda i: (0, 0))
    b_spec = pl.BlockSpec((1, DOUT), lambda i: (0, 0))
    o_spec = pl.BlockSpec((BR, DOUT), lambda i: (i, 0))
    return pl.pallas_call(
        _ka_body,
        grid=(NB,),
        in_specs=[row_spec, row_spec, row_spec, w_spec, b_spec,
                  w_spec, w_spec, b_spec],
        out_specs=[row_spec, o_spec, o_spec],
        out_shape=[jax.ShapeDtypeStruct((N, D), jnp.float32),
                   jax.ShapeDtypeStruct((N, DOUT), jnp.float32),
                   jax.ShapeDtypeStruct((N, DOUT), jnp.float32)],
    )(x, p0, p1, W1_0, b1, W2_0, W2_1, b2)


def _kb_body(x_ref, t1_ref, u0_ref, u1_ref, w30_ref, w31_ref, w32_ref, b3_ref,
             s3_ref):
    xb = x_ref[...]
    t1 = t1_ref[...]
    tx2 = 2.0 * (u0_ref[...] + u1_ref[...]) - xb
    s3_ref[...] = (jnp.dot(xb, w30_ref[...], preferred_element_type=jnp.float32)
                   + jnp.dot(t1, w31_ref[...], preferred_element_type=jnp.float32)
                   + jnp.dot(tx2, w32_ref[...], preferred_element_type=jnp.float32)
                   + b3_ref[...])


def _kb(x, tx1, u0, u1, W3_0, W3_1, W3_2, b3):
    row_spec = pl.BlockSpec((BR, D), lambda i: (i, 0))
    w_spec = pl.BlockSpec((D, DOUT), lambda i: (0, 0))
    b_spec = pl.BlockSpec((1, DOUT), lambda i: (0, 0))
    o_spec = pl.BlockSpec((BR, DOUT), lambda i: (i, 0))
    return pl.pallas_call(
        _kb_body,
        grid=(NB,),
        in_specs=[row_spec, row_spec, row_spec, row_spec,
                  w_spec, w_spec, w_spec, b_spec],
        out_specs=o_spec,
        out_shape=jax.ShapeDtypeStruct((N, DOUT), jnp.float32),
    )(x, tx1, u0, u1, W3_0, W3_1, W3_2, b3)


def kernel(x, edge_index, edge_weight, W1_0, b1, W2_0, W2_1, b2,
           W3_0, W3_1, W3_2, b3):
    rowc = edge_index[0].reshape(TOTCH, CH)
    colc = edge_index[1].reshape(TOTCH, CH)
    wc = edge_weight.reshape(TOTCH, CH)

    norm = _degnorm(rowc, colc, wc)
    nrmc = norm.reshape(TOTCH, CH)
    p = _prop(rowc, colc, nrmc, x)
    tx1, s1, s2 = _ka(x, p[0, :N], p[1, :N], W1_0, b1.reshape(1, DOUT),
                      W2_0, W2_1, b2.reshape(1, DOUT))
    u = _prop(rowc, colc, nrmc, tx1)
    s3 = _kb(x, tx1, u[0, :N], u[1, :N], W3_0, W3_1, W3_2, b3.reshape(1, DOUT))
    return s1, s2, s3


# speedup vs baseline: 19.5991x; 1.2523x over previous
import jax
import jax.numpy as jnp
from jax import lax
from jax.experimental import pallas as pl
from jax.experimental.pallas import tpu as pltpu
from jax.experimental.pallas import tpu_sc as plsc

N = 10000
NP = 10240
E = 320000
D = 128
DOUT = 200
NC = 2
NS = 16
NT = NC * NS
EPT = E // NT
CH = 80
NCHUNKS = EPT // CH
TOTCH = E // CH
DCH = TOTCH // NS
RPT = NP // NS
NB = 25
BR = N // NB


def _sc_mesh():
    return plsc.VectorSubcoreMesh(core_axis_name="c", subcore_axis_name="s")


def _degnorm_body(row_hbm, col_hbm, w_hbm, norm_hbm,
                  deg_sh, dis_sh, degv, nrm_st, t640, a640, d640,
                  sA, cA, wA, stsem):
    cid = lax.axis_index("c")
    sid = lax.axis_index("s")
    tid = cid * NS + sid

    def stage(eoff, half, ne):
        off = half * 4000
        eo = pl.multiple_of(eoff, 8)
        pltpu.async_copy(row_hbm.at[pl.ds(eo, ne)],
                         sA.at[pl.ds(off, ne)], stsem)
        pltpu.async_copy(col_hbm.at[pl.ds(eo, ne)],
                         cA.at[pl.ds(off, ne)], stsem)
        pltpu.async_copy(w_hbm.at[pl.ds(eo, ne)],
                         wA.at[pl.ds(off, ne)], stsem)

    def stage_wait(half, ne):
        off = half * 4000
        for _ in range(3):
            pltpu.make_async_copy(row_hbm.at[pl.ds(0, ne)],
                                  sA.at[pl.ds(off, ne)], stsem).wait()

    def zv(i, carry):
        degv[pl.ds(i * 16, 16)] = jnp.zeros((16,), jnp.float32)
        return carry

    lax.fori_loop(0, NP // 16, zv, 0)

    dbase = sid * DCH * CH
    stage(dbase, 0, 4000)
    for b in range(5):
        stage_wait(b % 2, 4000)
        if b + 1 < 5:
            stage(dbase + (b + 1) * 4000, (b + 1) % 2, 4000)
        off = (b % 2) * 4000

        def ablock(g, carry):
            sl = pl.ds(off + g * 16, 16)
            s16 = sA[sl]
            d16 = cA[sl]
            w16 = wA[sl]
            wm = jnp.where(s16 == d16, 0.0, w16)
            plsc.addupdate_scatter(degv, [s16], wm)
            return carry

        lax.fori_loop(0, 250, ablock, 0)
    pltpu.sync_copy(degv, deg_sh.at[sid])
    nbase = tid * EPT
    stage(nbase, 0, 2000)
    plsc.subcore_barrier()

    rbase = pl.multiple_of(sid * RPT, 8)
    pltpu.sync_copy(deg_sh.at[0, pl.ds(rbase, RPT)], a640)
    for t in range(1, NS):
        pltpu.sync_copy(deg_sh.at[t, pl.ds(rbase, RPT)], t640)

        def addb(g, carry):
            sl = pl.ds(g * 16, 16)
            a640[sl] = a640[sl] + t640[sl]
            return carry

        lax.fori_loop(0, RPT // 16, addb, 0)

    def disb(g, carry):
        sl = pl.ds(g * 16, 16)
        x16 = a640[sl]
        bits = plsc.bitcast(x16, jnp.int32)
        y = plsc.bitcast(jnp.int32(0x5F3759DF) - (bits >> 1), jnp.float32)
        for _ in range(3):
            y = y * (1.5 - 0.5 * x16 * y * y)
        d640[sl] = jnp.where(x16 > 0.0, y, 0.0)
        return carry

    lax.fori_loop(0, RPT // 16, disb, 0)
    pltpu.sync_copy(d640, dis_sh.at[pl.ds(rbase, RPT)])
    plsc.subcore_barrier()
    pltpu.sync_copy(dis_sh, degv)

    for b in range(5):
        stage_wait(b % 2, 2000)
        if b + 1 < 5:
            stage(nbase + (b + 1) * 2000, (b + 1) % 2, 2000)
        off = (b % 2) * 4000
        ebase = b * 2000

        def cblock(g, carry):
            sl = pl.ds(off + g * 16, 16)
            s16 = sA[sl]
            d16 = cA[sl]
            w16 = wA[sl]
            a16 = plsc.load_gather(degv, [s16])
            b16 = plsc.load_gather(degv, [d16])
            wm = jnp.where(s16 == d16, 0.0, w16)
            nrm_st[pl.ds(ebase + g * 16, 16)] = -(a16 * wm * b16)
            return carry

        lax.fori_loop(0, 125, cblock, 0)
    pltpu.sync_copy(nrm_st, norm_hbm.at[tid])


def _degnorm(row1, col1, w1):
    return pl.kernel(
        _degnorm_body,
        out_type=jax.ShapeDtypeStruct((NT, EPT), jnp.float32),
        mesh=_sc_mesh(),
        scratch_types=(
            [pltpu.VMEM_SHARED((NS, NP), jnp.float32),
             pltpu.VMEM_SHARED((NP,), jnp.float32),
             pltpu.VMEM((NP,), jnp.float32),
             pltpu.VMEM((EPT,), jnp.float32),
             pltpu.VMEM((RPT,), jnp.float32),
             pltpu.VMEM((RPT,), jnp.float32),
             pltpu.VMEM((RPT,), jnp.float32),
             pltpu.VMEM((8000,), jnp.int32),
             pltpu.VMEM((8000,), jnp.int32),
             pltpu.VMEM((8000,), jnp.float32),
             pltpu.SemaphoreType.DMA]
        ),
        compiler_params=pltpu.CompilerParams(needs_layout_passes=False),
    )(row1, col1, w1)


def _prop_body(row_hbm, col_hbm, nrm_hbm, x_hbm, out_hbm,
               acc_sh, rows0, rows1, rows2, siW, diW, nrW,
               gs0, gs1, gs2, ss0, ss1, ss2, stsem):
    cid = lax.axis_index("c")
    sid = lax.axis_index("s")
    tid = cid * NS + sid
    rows = (rows0, rows1, rows2)
    gsems = (gs0, gs1, gs2)
    ssems = (ss0, ss1, ss2)
    cbase = tid * NCHUNKS

    def zb(i, carry):
        for j in range(D // 16):
            rows0[i, pl.ds(j * 16, 16)] = jnp.zeros((16,), jnp.float32)
        return carry

    lax.fori_loop(0, CH, zb, 0)
    for b in range(RPT // CH):
        rb = pl.multiple_of(sid * RPT + b * CH, 8)
        pltpu.sync_copy(rows0, acc_sh.at[pl.ds(rb, CH)])

    def stage(blk, half):
        off = half * 2000
        eo = pl.multiple_of(cbase * CH + blk * 2000, 8)
        pltpu.async_copy(row_hbm.at[pl.ds(eo, 2000)],
                         siW.at[pl.ds(off, 2000)], stsem)
        pltpu.async_copy(col_hbm.at[pl.ds(eo, 2000)],
                         diW.at[pl.ds(off, 2000)], stsem)
        pltpu.async_copy(nrm_hbm.at[pl.ds(eo, 2000)],
                         nrW.at[pl.ds(off, 2000)], stsem)

    def stage_wait():
        for _ in range(3):
            pltpu.make_async_copy(row_hbm.at[pl.ds(0, 2000)],
                                  siW.at[pl.ds(0, 2000)], stsem).wait()

    stage(0, 0)
    stage_wait()
    plsc.subcore_barrier()

    def g_start(c, r3):
        ro = pl.multiple_of(lax.rem(c, 50) * CH, 8)
        pltpu.async_copy(x_hbm.at[siW.at[pl.ds(ro, CH)]], rows[r3],
                         gsems[r3])

    def g_wait(r3):
        pltpu.make_async_copy(x_hbm.at[siW.at[pl.ds(0, CH)]], rows[r3],
                              gsems[r3]).wait()

    def s_start(c, r3):
        ro = pl.multiple_of(lax.rem(c, 50) * CH, 8)
        pltpu.async_copy(rows[r3], acc_sh.at[diW.at[pl.ds(ro, CH)]],
                         ssems[r3], add=True)

    def s_wait(r3):
        pltpu.make_async_copy(rows[0], acc_sh.at[diW.at[pl.ds(0, CH)]],
                              ssems[r3]).wait()

    def scale(c, r3):
        buf = rows[r3]
        ro = pl.multiple_of(lax.rem(c, 50) * CH, 8)

        def kbody(k, carry):
            n16 = nrW[pl.ds(ro + k * 16, 16)]
            e0 = k * 16
            for l in range(16):
                nv = n16[l]
                for j in range(D // 16):
                    fs = pl.ds(j * 16, 16)
                    buf[e0 + l, fs] = buf[e0 + l, fs] * nv
            return carry

        lax.fori_loop(0, CH // 16, kbody, 0)

    def step(c, r3, do_swait, do_g):
        g_wait(r3)
        scale(c, r3)
        s_start(c, r3)
        if do_swait:
            s_wait((r3 + 2) % 3)
        blk = c // 25 if isinstance(c, int) else lax.div(c, 25)
        q = c % 25 if isinstance(c, int) else lax.rem(c, 25)
        nxt_needs_stage = (q == 0) & (blk < 4) if not isinstance(c, int) \
            else (q == 0 and blk < 4)

        def do_stage():
            stage(blk + 1, lax.rem(blk + 1, 2) if not isinstance(c, int)
                  else (blk + 1) % 2)

        if isinstance(c, int):
            if nxt_needs_stage:
                do_stage()
        else:
            pl.when(nxt_needs_stage)(do_stage)

        q2 = (c + 2) % 25 if isinstance(c, int) else lax.rem(c + 2, 25)
        b2 = (c + 2) // 25 if isinstance(c, int) else lax.div(c + 2, 25)
        wait_needed = (q2 == 0) & (b2 <= 4) if not isinstance(c, int) \
            else (q2 == 0 and b2 <= 4)
        if isinstance(c, int):
            if wait_needed:
                stage_wait()
        else:
            pl.when(wait_needed)(stage_wait)
        if do_g:
            g_start(c + 2, (r3 + 2) % 3)

    g_start(0, 0)
    g_start(1, 1)
    step(0, 0, False, True)

    def triple(pp, carry):
        c = 1 + 3 * pp
        for q in range(3):
            step(c + q, (1 + q) % 3, True, True)
        return carry

    lax.fori_loop(0, 40, triple, 0)
    for c in range(121, NCHUNKS):
        step(c, c % 3, True, c + 2 < NCHUNKS)
    s_wait((NCHUNKS - 1) % 3)
    plsc.subcore_barrier()
    for b in range(RPT // 128):
        rb = pl.multiple_of(sid * RPT + b * 128, 8)
        pltpu.async_copy(acc_sh.at[pl.ds(rb, 128)],
                         out_hbm.at[cid, pl.ds(rb, 128)], stsem)
    for b in range(RPT // 128):
        rb = pl.multiple_of(sid * RPT + b * 128, 8)
        pltpu.make_async_copy(acc_sh.at[pl.ds(rb, 128)],
                              out_hbm.at[cid, pl.ds(rb, 128)], stsem).wait()


def _prop(row1, col1, nrm1, xsrc):
    return pl.kernel(
        _prop_body,
        out_type=jax.ShapeDtypeStruct((NC, NP, D), jnp.float32),
        mesh=_sc_mesh(),
        scratch_types=(
            [pltpu.VMEM_SHARED((NP, D), jnp.float32)]
            + [pltpu.VMEM((CH, D), jnp.float32) for _ in range(3)]
            + [pltpu.VMEM((4000,), jnp.int32) for _ in range(2)]
            + [pltpu.VMEM((4000,), jnp.float32)]
            + [pltpu.SemaphoreType.DMA for _ in range(7)]
        ),
        compiler_params=pltpu.CompilerParams(needs_layout_passes=False),
    )(row1, col1, nrm1, xsrc)


def _ka_body(x_ref, p0_ref, p1_ref, w10_ref, b1_ref, w20_ref, w21_ref, b2_ref,
             tx1_ref, s1_ref, s2_ref):
    t1 = p0_ref[...] + p1_ref[...]
    tx1_ref[...] = t1
    xb = x_ref[...]
    s1_ref[...] = jnp.dot(xb, w10_ref[...],
                          preferred_element_type=jnp.float32) + b1_ref[...]
    s2_ref[...] = (jnp.dot(xb, w20_ref[...], preferred_element_type=jnp.float32)
                   + jnp.dot(t1, w21_ref[...], preferred_element_type=jnp.float32)
                   + b2_ref[...])


def _ka(x, p0, p1, W1_0, b1, W2_0, W2_1, b2):
    row_spec = pl.BlockSpec((BR, D), lambda i: (i, 0))
    w_spec = pl.BlockSpec((D, DOUT), lambda i: (0, 0))
    b_spec = pl.BlockSpec((1, DOUT), lambda i: (0, 0))
    o_spec = pl.BlockSpec((BR, DOUT), lambda i: (i, 0))
    return pl.pallas_call(
        _ka_body,
        grid=(NB,),
        in_specs=[row_spec, row_spec, row_spec, w_spec, b_spec,
                  w_spec, w_spec, b_spec],
        out_specs=[row_spec, o_spec, o_spec],
        out_shape=[jax.ShapeDtypeStruct((N, D), jnp.float32),
                   jax.ShapeDtypeStruct((N, DOUT), jnp.float32),
                   jax.ShapeDtypeStruct((N, DOUT), jnp.float32)],
    )(x, p0, p1, W1_0, b1, W2_0, W2_1, b2)


def _kb_body(x_ref, t1_ref, u0_ref, u1_ref, w30_ref, w31_ref, w32_ref, b3_ref,
             s3_ref):
    xb = x_ref[...]
    t1 = t1_ref[...]
    tx2 = 2.0 * (u0_ref[...] + u1_ref[...]) - xb
    s3_ref[...] = (jnp.dot(xb, w30_ref[...], preferred_element_type=jnp.float32)
                   + jnp.dot(t1, w31_ref[...], preferred_element_type=jnp.float32)
                   + jnp.dot(tx2, w32_ref[...], preferred_element_type=jnp.float32)
                   + b3_ref[...])


def _kb(x, tx1, u0, u1, W3_0, W3_1, W3_2, b3):
    row_spec = pl.BlockSpec((BR, D), lambda i: (i, 0))
    w_spec = pl.BlockSpec((D, DOUT), lambda i: (0, 0))
    b_spec = pl.BlockSpec((1, DOUT), lambda i: (0, 0))
    o_spec = pl.BlockSpec((BR, DOUT), lambda i: (i, 0))
    return pl.pallas_call(
        _kb_body,
        grid=(NB,),
        in_specs=[row_spec, row_spec, row_spec, row_spec,
                  w_spec, w_spec, w_spec, b_spec],
        out_specs=o_spec,
        out_shape=jax.ShapeDtypeStruct((N, DOUT), jnp.float32),
    )(x, tx1, u0, u1, W3_0, W3_1, W3_2, b3)


def kernel(x, edge_index, edge_weight, W1_0, b1, W2_0, W2_1, b2,
           W3_0, W3_1, W3_2, b3):
    row1 = edge_index[0]
    col1 = edge_index[1]

    norm = _degnorm(row1, col1, edge_weight)
    nrm1 = norm.reshape(E)
    p = _prop(row1, col1, nrm1, x)
    tx1, s1, s2 = _ka(x, p[0, :N], p[1, :N], W1_0, b1.reshape(1, DOUT),
                      W2_0, W2_1, b2.reshape(1, DOUT))
    u = _prop(row1, col1, nrm1, tx1)
    s3 = _kb(x, tx1, u[0, :N], u[1, :N], W3_0, W3_1, W3_2, b3.reshape(1, DOUT))
    return s1, s2, s3


# speedup vs baseline: 19.6849x; 1.0044x over previous
import jax
import jax.numpy as jnp
from jax import lax
from jax.experimental import pallas as pl
from jax.experimental.pallas import tpu as pltpu
from jax.experimental.pallas import tpu_sc as plsc

N = 10000
NP = 10240
E = 320000
D = 128
DOUT = 200
NC = 2
NS = 16
NT = NC * NS
EPT = E // NT
CH = 80
NCHUNKS = EPT // CH
TOTCH = E // CH
DCH = TOTCH // NS
RPT = NP // NS
NB = 25
BR = N // NB


def _sc_mesh():
    return plsc.VectorSubcoreMesh(core_axis_name="c", subcore_axis_name="s")


def _degnorm_body(row_hbm, col_hbm, w_hbm, norm_hbm,
                  deg_sh, dis_sh, degv, nrm_st, t640, a640, d640,
                  sA, cA, wA, stsem):
    cid = lax.axis_index("c")
    sid = lax.axis_index("s")
    tid = cid * NS + sid

    def stage(eoff, half, ne):
        off = half * 4000
        eo = pl.multiple_of(eoff, 8)
        pltpu.async_copy(row_hbm.at[pl.ds(eo, ne)],
                         sA.at[pl.ds(off, ne)], stsem)
        pltpu.async_copy(col_hbm.at[pl.ds(eo, ne)],
                         cA.at[pl.ds(off, ne)], stsem)
        pltpu.async_copy(w_hbm.at[pl.ds(eo, ne)],
                         wA.at[pl.ds(off, ne)], stsem)

    def stage_wait(half, ne):
        off = half * 4000
        for _ in range(3):
            pltpu.make_async_copy(row_hbm.at[pl.ds(0, ne)],
                                  sA.at[pl.ds(off, ne)], stsem).wait()

    def zv(i, carry):
        degv[pl.ds(i * 16, 16)] = jnp.zeros((16,), jnp.float32)
        return carry

    lax.fori_loop(0, NP // 16, zv, 0)

    dbase = sid * DCH * CH
    stage(dbase, 0, 4000)
    for b in range(5):
        stage_wait(b % 2, 4000)
        if b + 1 < 5:
            stage(dbase + (b + 1) * 4000, (b + 1) % 2, 4000)
        off = (b % 2) * 4000

        def ablock(g, carry):
            sl = pl.ds(off + g * 16, 16)
            s16 = sA[sl]
            d16 = cA[sl]
            w16 = wA[sl]
            wm = jnp.where(s16 == d16, 0.0, w16)
            plsc.addupdate_scatter(degv, [s16], wm)
            return carry

        lax.fori_loop(0, 250, ablock, 0)
    pltpu.sync_copy(degv, deg_sh.at[sid])
    nbase = tid * EPT
    stage(nbase, 0, 2000)
    plsc.subcore_barrier()

    rbase = pl.multiple_of(sid * RPT, 8)
    pltpu.sync_copy(deg_sh.at[0, pl.ds(rbase, RPT)], a640)
    for t in range(1, NS):
        pltpu.sync_copy(deg_sh.at[t, pl.ds(rbase, RPT)], t640)

        def addb(g, carry):
            sl = pl.ds(g * 16, 16)
            a640[sl] = a640[sl] + t640[sl]
            return carry

        lax.fori_loop(0, RPT // 16, addb, 0)

    def disb(g, carry):
        sl = pl.ds(g * 16, 16)
        x16 = a640[sl]
        bits = plsc.bitcast(x16, jnp.int32)
        y = plsc.bitcast(jnp.int32(0x5F3759DF) - (bits >> 1), jnp.float32)
        for _ in range(3):
            y = y * (1.5 - 0.5 * x16 * y * y)
        d640[sl] = jnp.where(x16 > 0.0, y, 0.0)
        return carry

    lax.fori_loop(0, RPT // 16, disb, 0)
    pltpu.sync_copy(d640, dis_sh.at[pl.ds(rbase, RPT)])
    plsc.subcore_barrier()
    pltpu.sync_copy(dis_sh, degv)

    for b in range(5):
        stage_wait(b % 2, 2000)
        if b + 1 < 5:
            stage(nbase + (b + 1) * 2000, (b + 1) % 2, 2000)
        off = (b % 2) * 4000
        ebase = b * 2000

        def cblock(g, carry):
            sl = pl.ds(off + g * 16, 16)
            s16 = sA[sl]
            d16 = cA[sl]
            w16 = wA[sl]
            a16 = plsc.load_gather(degv, [s16])
            b16 = plsc.load_gather(degv, [d16])
            wm = jnp.where(s16 == d16, 0.0, w16)
            nrm_st[pl.ds(ebase + g * 16, 16)] = -(a16 * wm * b16)
            return carry

        lax.fori_loop(0, 125, cblock, 0)
    pltpu.sync_copy(nrm_st, norm_hbm.at[tid])


def _degnorm(row1, col1, w1):
    return pl.kernel(
        _degnorm_body,
        out_type=jax.ShapeDtypeStruct((NT, EPT), jnp.float32),
        mesh=_sc_mesh(),
        scratch_types=(
            [pltpu.VMEM_SHARED((NS, NP), jnp.float32),
             pltpu.VMEM_SHARED((NP,), jnp.float32),
             pltpu.VMEM((NP,), jnp.float32),
             pltpu.VMEM((EPT,), jnp.float32),
             pltpu.VMEM((RPT,), jnp.float32),
             pltpu.VMEM((RPT,), jnp.float32),
             pltpu.VMEM((RPT,), jnp.float32),
             pltpu.VMEM((8000,), jnp.int32),
             pltpu.VMEM((8000,), jnp.int32),
             pltpu.VMEM((8000,), jnp.float32),
             pltpu.SemaphoreType.DMA]
        ),
        compiler_params=pltpu.CompilerParams(needs_layout_passes=False),
    )(row1, col1, w1)


def _prop_body(row_hbm, col_hbm, nrm_hbm, x_hbm, out_hbm,
               acc_sh, rows0, rows1, rows2, siW, diW, nrW,
               gs0, gs1, gs2, ss0, ss1, ss2, stsem):
    cid = lax.axis_index("c")
    sid = lax.axis_index("s")
    tid = cid * NS + sid
    rows = (rows0, rows1, rows2)
    gsems = (gs0, gs1, gs2)
    ssems = (ss0, ss1, ss2)
    cbase = tid * NCHUNKS

    def zb(i, carry):
        for j in range(D // 16):
            rows0[i, pl.ds(j * 16, 16)] = jnp.zeros((16,), jnp.float32)
        return carry

    lax.fori_loop(0, CH, zb, 0)
    for b in range(RPT // CH):
        rb = pl.multiple_of(sid * RPT + b * CH, 8)
        pltpu.sync_copy(rows0, acc_sh.at[pl.ds(rb, CH)])

    def stage(blk, half):
        off = half * 2000
        eo = pl.multiple_of(cbase * CH + blk * 2000, 8)
        pltpu.async_copy(row_hbm.at[pl.ds(eo, 2000)],
                         siW.at[pl.ds(off, 2000)], stsem)
        pltpu.async_copy(col_hbm.at[pl.ds(eo, 2000)],
                         diW.at[pl.ds(off, 2000)], stsem)
        pltpu.async_copy(nrm_hbm.at[pl.ds(eo, 2000)],
                         nrW.at[pl.ds(off, 2000)], stsem)

    def stage_wait():
        for _ in range(3):
            pltpu.make_async_copy(row_hbm.at[pl.ds(0, 2000)],
                                  siW.at[pl.ds(0, 2000)], stsem).wait()

    stage(0, 0)
    stage_wait()
    plsc.subcore_barrier()

    def g_start(c, r3):
        ro = pl.multiple_of(lax.rem(c, 50) * CH, 8)
        pltpu.async_copy(x_hbm.at[siW.at[pl.ds(ro, CH)]], rows[r3],
                         gsems[r3])

    def g_wait(r3):
        pltpu.make_async_copy(x_hbm.at[siW.at[pl.ds(0, CH)]], rows[r3],
                              gsems[r3]).wait()

    def s_start(c, r3):
        ro = pl.multiple_of(lax.rem(c, 50) * CH, 8)
        pltpu.async_copy(rows[r3], acc_sh.at[diW.at[pl.ds(ro, CH)]],
                         ssems[r3], add=True)

    def s_wait(r3):
        pltpu.make_async_copy(rows[0], acc_sh.at[diW.at[pl.ds(0, CH)]],
                              ssems[r3]).wait()

    def scale(c, r3):
        buf = rows[r3]
        ro = pl.multiple_of(lax.rem(c, 50) * CH, 8)

        def kbody(k, carry):
            n16 = nrW[pl.ds(ro + k * 16, 16)]
            e0 = k * 16
            for l in range(16):
                nv = n16[l]
                for j in range(D // 16):
                    fs = pl.ds(j * 16, 16)
                    buf[e0 + l, fs] = buf[e0 + l, fs] * nv
            return carry

        lax.fori_loop(0, CH // 16, kbody, 0)

    def step(c, r3, do_swait, do_g):
        g_wait(r3)
        scale(c, r3)
        s_start(c, r3)
        if do_swait:
            s_wait((r3 + 2) % 3)
        blk = c // 25 if isinstance(c, int) else lax.div(c, 25)
        q = c % 25 if isinstance(c, int) else lax.rem(c, 25)
        nxt_needs_stage = (q == 0) & (blk < 4) if not isinstance(c, int) \
            else (q == 0 and blk < 4)

        def do_stage():
            stage(blk + 1, lax.rem(blk + 1, 2) if not isinstance(c, int)
                  else (blk + 1) % 2)

        if isinstance(c, int):
            if nxt_needs_stage:
                do_stage()
        else:
            pl.when(nxt_needs_stage)(do_stage)

        q2 = (c + 2) % 25 if isinstance(c, int) else lax.rem(c + 2, 25)
        b2 = (c + 2) // 25 if isinstance(c, int) else lax.div(c + 2, 25)
        wait_needed = (q2 == 0) & (b2 <= 4) if not isinstance(c, int) \
            else (q2 == 0 and b2 <= 4)
        if isinstance(c, int):
            if wait_needed:
                stage_wait()
        else:
            pl.when(wait_needed)(stage_wait)
        if do_g:
            g_start(c + 2, (r3 + 2) % 3)

    g_start(0, 0)
    g_start(1, 1)
    step(0, 0, False, True)

    def triple(pp, carry):
        c = 1 + 3 * pp
        for q in range(3):
            step(c + q, (1 + q) % 3, True, True)
        return carry

    lax.fori_loop(0, 40, triple, 0)
    for c in range(121, NCHUNKS):
        step(c, c % 3, True, c + 2 < NCHUNKS)
    s_wait((NCHUNKS - 1) % 3)
    plsc.subcore_barrier()
    for b in range(RPT // 128):
        rb = pl.multiple_of(sid * RPT + b * 128, 8)
        pltpu.async_copy(acc_sh.at[pl.ds(rb, 128)],
                         out_hbm.at[cid, pl.ds(rb, 128)], stsem)
    for b in range(RPT // 128):
        rb = pl.multiple_of(sid * RPT + b * 128, 8)
        pltpu.make_async_copy(acc_sh.at[pl.ds(rb, 128)],
                              out_hbm.at[cid, pl.ds(rb, 128)], stsem).wait()


def _prop(row1, col1, nrm1, xsrc):
    return pl.kernel(
        _prop_body,
        out_type=jax.ShapeDtypeStruct((NC, NP, D), jnp.float32),
        mesh=_sc_mesh(),
        scratch_types=(
            [pltpu.VMEM_SHARED((NP, D), jnp.float32)]
            + [pltpu.VMEM((CH, D), jnp.float32) for _ in range(3)]
            + [pltpu.VMEM((4000,), jnp.int32) for _ in range(2)]
            + [pltpu.VMEM((4000,), jnp.float32)]
            + [pltpu.SemaphoreType.DMA for _ in range(7)]
        ),
        compiler_params=pltpu.CompilerParams(needs_layout_passes=False),
    )(row1, col1, nrm1, xsrc)


def _k1_body(x_ref, w10_ref, b1_ref, s1_ref):
    s1_ref[...] = jnp.dot(x_ref[...], w10_ref[...],
                          preferred_element_type=jnp.float32) + b1_ref[...]


def _k1(x, W1_0, b1):
    return pl.pallas_call(
        _k1_body,
        grid=(NB,),
        in_specs=[pl.BlockSpec((BR, D), lambda i: (i, 0)),
                  pl.BlockSpec((D, DOUT), lambda i: (0, 0)),
                  pl.BlockSpec((1, DOUT), lambda i: (0, 0))],
        out_specs=pl.BlockSpec((BR, DOUT), lambda i: (i, 0)),
        out_shape=jax.ShapeDtypeStruct((N, DOUT), jnp.float32),
    )(x, W1_0, b1)


def _ka_body(x_ref, p0_ref, p1_ref, w20_ref, w21_ref, b2_ref,
             tx1_ref, s2_ref):
    t1 = p0_ref[...] + p1_ref[...]
    tx1_ref[...] = t1
    s2_ref[...] = (jnp.dot(x_ref[...], w20_ref[...],
                           preferred_element_type=jnp.float32)
                   + jnp.dot(t1, w21_ref[...], preferred_element_type=jnp.float32)
                   + b2_ref[...])


def _ka(x, p0, p1, W2_0, W2_1, b2):
    row_spec = pl.BlockSpec((BR, D), lambda i: (i, 0))
    w_spec = pl.BlockSpec((D, DOUT), lambda i: (0, 0))
    b_spec = pl.BlockSpec((1, DOUT), lambda i: (0, 0))
    o_spec = pl.BlockSpec((BR, DOUT), lambda i: (i, 0))
    return pl.pallas_call(
        _ka_body,
        grid=(NB,),
        in_specs=[row_spec, row_spec, row_spec, w_spec, w_spec, b_spec],
        out_specs=[row_spec, o_spec],
        out_shape=[jax.ShapeDtypeStruct((N, D), jnp.float32),
                   jax.ShapeDtypeStruct((N, DOUT), jnp.float32)],
    )(x, p0, p1, W2_0, W2_1, b2)


def _kb_body(x_ref, t1_ref, u0_ref, u1_ref, w30_ref, w31_ref, w32_ref, b3_ref,
             s3_ref):
    xb = x_ref[...]
    t1 = t1_ref[...]
    tx2 = 2.0 * (u0_ref[...] + u1_ref[...]) - xb
    s3_ref[...] = (jnp.dot(xb, w30_ref[...], preferred_element_type=jnp.float32)
                   + jnp.dot(t1, w31_ref[...], preferred_element_type=jnp.float32)
                   + jnp.dot(tx2, w32_ref[...], preferred_element_type=jnp.float32)
                   + b3_ref[...])


def _kb(x, tx1, u0, u1, W3_0, W3_1, W3_2, b3):
    row_spec = pl.BlockSpec((BR, D), lambda i: (i, 0))
    w_spec = pl.BlockSpec((D, DOUT), lambda i: (0, 0))
    b_spec = pl.BlockSpec((1, DOUT), lambda i: (0, 0))
    o_spec = pl.BlockSpec((BR, DOUT), lambda i: (i, 0))
    return pl.pallas_call(
        _kb_body,
        grid=(NB,),
        in_specs=[row_spec, row_spec, row_spec, row_spec,
                  w_spec, w_spec, w_spec, b_spec],
        out_specs=o_spec,
        out_shape=jax.ShapeDtypeStruct((N, DOUT), jnp.float32),
    )(x, tx1, u0, u1, W3_0, W3_1, W3_2, b3)


def kernel(x, edge_index, edge_weight, W1_0, b1, W2_0, W2_1, b2,
           W3_0, W3_1, W3_2, b3):
    row1 = edge_index[0]
    col1 = edge_index[1]

    s1 = _k1(x, W1_0, b1.reshape(1, DOUT))
    norm = _degnorm(row1, col1, edge_weight)
    nrm1 = norm.reshape(E)
    p = _prop(row1, col1, nrm1, x)
    tx1, s2 = _ka(x, p[0, :N], p[1, :N], W2_0, W2_1, b2.reshape(1, DOUT))
    u = _prop(row1, col1, nrm1, tx1)
    s3 = _kb(x, tx1, u[0, :N], u[1, :N], W3_0, W3_1, W3_2, b3.reshape(1, DOUT))
    return s1, s2, s3
